# Initial kernel scaffold; baseline (speedup 1.0000x reference)
import jax
import jax.numpy as jnp
from jax.experimental import pallas as pl


def kernel(x, edge_index, edge_attr, lin_in_W, lin_in_b, mlp0_W1, mlp0_b1, mlp0_W2, mlp0_b2, root0, bias0, mlp1_W1, mlp1_b1, mlp1_W2, mlp1_b2, root1, bias1, mlp2_W1, mlp2_b1, mlp2_W2, mlp2_b2, root2, bias2, lin_out_W, lin_out_b):
    raise NotImplementedError("write your pallas kernel here")


# speedup vs baseline: 1.9687x; 1.9687x over previous
import functools

import jax
import jax.numpy as jnp
from jax import lax
from jax.experimental import pallas as pl
from jax.experimental.pallas import tpu as pltpu
from jax.experimental.pallas import tpu_sc as plsc

_NC = 2
_NS = 16
_NW = _NC * _NS
_CH = 128


def _sc_mesh():
    return plsc.VectorSubcoreMesh(core_axis_name="c", subcore_axis_name="s",
                                  num_cores=_NC, num_subcores=_NS)


_SC_PARAMS = pltpu.CompilerParams(use_tc_tiling_on_sc=False)


def _make_gather(N, E, H):
    epw = E // _NW
    nfull = epw // _CH
    tail = epw - nfull * _CH
    gf = 13 if nfull % 13 == 0 else 1

    @functools.partial(
        pl.kernel,
        mesh=_sc_mesh(),
        out_type=jax.ShapeDtypeStruct((E, H), jnp.float32),
        scratch_types=[
            pltpu.VMEM((epw,), jnp.int32),
            pltpu.VMEM((epw, H), jnp.float32),
            pltpu.SemaphoreType.DMA,
        ],
        compiler_params=_SC_PARAMS,
    )
    def gather_k(h_hbm, src_hbm, out_hbm, idx_v, rows_v, sem):
        wid = lax.axis_index("s") * _NC + lax.axis_index("c")
        base = wid * epw
        pltpu.sync_copy(src_hbm.at[pl.ds(base, epw)], idx_v)

        @pl.loop(0, nfull // gf)
        def _(g):
            descs = []
            for i in range(gf):
                off = (g * gf + i) * _CH
                descs.append(pltpu.async_copy(
                    h_hbm.at[idx_v.at[pl.ds(off, _CH)]],
                    rows_v.at[pl.ds(off, _CH)], sem))
            for d in descs:
                d.wait()

        if tail:
            off = nfull * _CH
            pltpu.async_copy(h_hbm.at[idx_v.at[pl.ds(off, tail)]],
                             rows_v.at[pl.ds(off, tail)], sem).wait()
        pltpu.sync_copy(rows_v, out_hbm.at[pl.ds(base, epw)])

    return gather_k


def _make_scatter(N, E, H):
    epw = E // _NW
    nfull = epw // _CH
    tail = epw - nfull * _CH
    rps = N // _NS

    @functools.partial(
        pl.kernel,
        mesh=_sc_mesh(),
        out_type=jax.ShapeDtypeStruct((_NC, N, H), jnp.float32),
        scratch_types=[
            pltpu.VMEM((epw, H), jnp.float32),
            pltpu.VMEM((nfull, _CH), jnp.int32),
            pltpu.VMEM((tail,), jnp.int32) if tail else None,
            pltpu.VMEM_SHARED((N, H), jnp.float32),
        ],
        compiler_params=_SC_PARAMS,
    )
    def scatter_k(msg_hbm, dst_hbm, out_hbm, vals_v, idx_v, idxt_v, acc_sh):
        cid = lax.axis_index("c")
        sid = lax.axis_index("s")
        wid = sid * _NC + cid
        base = wid * epw

        @pl.loop(0, rps)
        def _(i):
            vals_v[i, :] = jnp.zeros((H,), jnp.float32)

        pltpu.sync_copy(vals_v.at[pl.ds(0, rps)],
                        acc_sh.at[pl.ds(sid * rps, rps)])
        plsc.subcore_barrier()

        pltpu.sync_copy(msg_hbm.at[pl.ds(base, epw)], vals_v)

        @pl.loop(0, nfull)
        def _(j):
            pltpu.sync_copy(dst_hbm.at[pl.ds(base + j * _CH, _CH)],
                            idx_v.at[j])

        if tail:
            pltpu.sync_copy(dst_hbm.at[pl.ds(base + nfull * _CH, tail)],
                            idxt_v)

        @pl.loop(0, nfull)
        def _(j):
            pltpu.sync_copy(vals_v.at[pl.ds(j * _CH, _CH)],
                            acc_sh.at[idx_v.at[j]], add=True)

        if tail:
            pltpu.sync_copy(vals_v.at[pl.ds(nfull * _CH, tail)],
                            acc_sh.at[idxt_v], add=True)
        plsc.subcore_barrier()
        pltpu.sync_copy(acc_sh.at[pl.ds(sid * rps, rps)],
                        out_hbm.at[cid, pl.ds(sid * rps, rps)])

    return scatter_k


def _make_counts(N, E, H):
    epw = E // _NW
    nfull = epw // _CH
    tail = epw - nfull * _CH
    rps = N // _NS

    @functools.partial(
        pl.kernel,
        mesh=_sc_mesh(),
        out_type=jax.ShapeDtypeStruct((_NC, N, H), jnp.float32),
        scratch_types=[
            pltpu.VMEM((max(_CH, rps), H), jnp.float32),
            pltpu.VMEM((nfull, _CH), jnp.int32),
            pltpu.VMEM((tail,), jnp.int32) if tail else None,
            pltpu.VMEM_SHARED((N, H), jnp.float32),
        ],
        compiler_params=_SC_PARAMS,
    )
    def counts_k(dst_hbm, out_hbm, ones_v, idx_v, idxt_v, acc_sh):
        cid = lax.axis_index("c")
        sid = lax.axis_index("s")
        wid = sid * _NC + cid
        base = wid * epw

        @pl.loop(0, max(_CH, rps))
        def _(i):
            ones_v[i, :] = jnp.zeros((H,), jnp.float32)

        pltpu.sync_copy(ones_v.at[pl.ds(0, rps)],
                        acc_sh.at[pl.ds(sid * rps, rps)])
        plsc.subcore_barrier()

        @pl.loop(0, _CH)
        def _(i):
            ones_v[i, :] = jnp.ones((H,), jnp.float32)

        @pl.loop(0, nfull)
        def _(j):
            pltpu.sync_copy(dst_hbm.at[pl.ds(base + j * _CH, _CH)],
                            idx_v.at[j])

        if tail:
            pltpu.sync_copy(dst_hbm.at[pl.ds(base + nfull * _CH, tail)],
                            idxt_v)

        @pl.loop(0, nfull)
        def _(j):
            pltpu.sync_copy(ones_v.at[pl.ds(0, _CH)],
                            acc_sh.at[idx_v.at[j]], add=True)

        if tail:
            pltpu.sync_copy(ones_v.at[pl.ds(0, tail)],
                            acc_sh.at[idxt_v], add=True)
        plsc.subcore_barrier()
        pltpu.sync_copy(acc_sh.at[pl.ds(sid * rps, rps)],
                        out_hbm.at[cid, pl.ds(sid * rps, rps)])

    return counts_k


def _pre_proj(x, W, b, block=2000):
    N, F = x.shape
    H = W.shape[1]

    def body(x_ref, w_ref, b_ref, o_ref):
        o_ref[...] = jnp.maximum(
            jnp.dot(x_ref[...], w_ref[...],
                    preferred_element_type=jnp.float32) + b_ref[...], 0.0)

    return pl.pallas_call(
        body,
        grid=(N // block,),
        in_specs=[
            pl.BlockSpec((block, F), lambda i: (i, 0)),
            pl.BlockSpec((F, H), lambda i: (0, 0)),
            pl.BlockSpec((1, H), lambda i: (0, 0)),
        ],
        out_specs=pl.BlockSpec((block, H), lambda i: (i, 0)),
        out_shape=jax.ShapeDtypeStruct((N, H), jnp.float32),
        compiler_params=pltpu.CompilerParams(
            dimension_semantics=("parallel",)),
    )(x, W, b.reshape(1, H))


def _messages(ea, hs, W1, b1, W2s, B2, block=2000):
    E, FE = ea.shape
    H = hs.shape[1]

    def body(ea_ref, hs_ref, w1_ref, b1_ref, w2_ref, b2_ref, o_ref):
        t = jnp.maximum(
            jnp.dot(ea_ref[...], w1_ref[...],
                    preferred_element_type=jnp.float32) + b1_ref[...], 0.0)
        hs_b = hs_ref[...]
        acc = jnp.dot(hs_b, b2_ref[...], preferred_element_type=jnp.float32)
        for k in range(H):
            acc = acc + t[:, k:k + 1] * jnp.dot(
                hs_b, w2_ref[k], preferred_element_type=jnp.float32)
        o_ref[...] = acc

    return pl.pallas_call(
        body,
        grid=(E // block,),
        in_specs=[
            pl.BlockSpec((block, FE), lambda i: (i, 0)),
            pl.BlockSpec((block, H), lambda i: (i, 0)),
            pl.BlockSpec((FE, H), lambda i: (0, 0)),
            pl.BlockSpec((1, H), lambda i: (0, 0)),
            pl.BlockSpec((FE, H, H), lambda i: (0, 0, 0)),
            pl.BlockSpec((H, H), lambda i: (0, 0)),
        ],
        out_specs=pl.BlockSpec((block, H), lambda i: (i, 0)),
        out_shape=jax.ShapeDtypeStruct((E, H), jnp.float32),
        compiler_params=pltpu.CompilerParams(
            dimension_semantics=("parallel",)),
    )(ea, hs, W1, b1.reshape(1, H), W2s, B2)


def _node_update(s0, s1, c0, c1, h, root, bias):
    N, H = h.shape

    def body(s0_ref, s1_ref, c0_ref, c1_ref, h_ref, r_ref, b_ref, o_ref):
        s = s0_ref[...] + s1_ref[...]
        cnt = jnp.maximum(c0_ref[...] + c1_ref[...], 1.0)
        o_ref[...] = jnp.maximum(
            s / cnt + jnp.dot(h_ref[...], r_ref[...],
                              preferred_element_type=jnp.float32) + b_ref[...],
            0.0)

    return pl.pallas_call(
        body,
        out_shape=jax.ShapeDtypeStruct((N, H), jnp.float32),
    )(s0, s1, c0, c1, h, root, bias.reshape(1, H))


def _node_update_out(s0, s1, c0, c1, h, root, bias, Wout, bout):
    N, H = h.shape
    OUT = Wout.shape[1]

    def body(s0_ref, s1_ref, c0_ref, c1_ref, h_ref, r_ref, b_ref,
             wo_ref, bo_ref, o_ref):
        s = s0_ref[...] + s1_ref[...]
        cnt = jnp.maximum(c0_ref[...] + c1_ref[...], 1.0)
        hn = jnp.maximum(
            s / cnt + jnp.dot(h_ref[...], r_ref[...],
                              preferred_element_type=jnp.float32) + b_ref[...],
            0.0)
        o_ref[...] = jnp.dot(hn, wo_ref[...],
                             preferred_element_type=jnp.float32) + bo_ref[...]

    return pl.pallas_call(
        body,
        out_shape=jax.ShapeDtypeStruct((N, OUT), jnp.float32),
    )(s0, s1, c0, c1, h, root, bias.reshape(1, H), Wout, bout.reshape(1, OUT))


def kernel(x, edge_index, edge_attr, lin_in_W, lin_in_b,
           mlp0_W1, mlp0_b1, mlp0_W2, mlp0_b2, root0, bias0,
           mlp1_W1, mlp1_b1, mlp1_W2, mlp1_b2, root1, bias1,
           mlp2_W1, mlp2_b1, mlp2_W2, mlp2_b2, root2, bias2,
           lin_out_W, lin_out_b):
    N, _ = x.shape
    E = edge_index.shape[1]
    H = lin_in_W.shape[1]

    src = edge_index[0]
    dst = edge_index[1]

    gather_k = _make_gather(N, E, H)
    scatter_k = _make_scatter(N, E, H)
    counts_k = _make_counts(N, E, H)

    layer_params = [(mlp0_W1, mlp0_b1, mlp0_W2, mlp0_b2, root0, bias0),
                    (mlp1_W1, mlp1_b1, mlp1_W2, mlp1_b2, root1, bias1),
                    (mlp2_W1, mlp2_b1, mlp2_W2, mlp2_b2, root2, bias2)]

    h = _pre_proj(x, lin_in_W, lin_in_b)
    cnt = counts_k(dst)
    c0, c1 = cnt[0], cnt[1]

    out = None
    for l, (W1, b1, W2, b2, root, bias) in enumerate(layer_params):
        hs = gather_k(h, src)
        msg = _messages(edge_attr, hs, W1, b1, W2.reshape(H, H, H),
                        b2.reshape(H, H))
        s = scatter_k(msg, dst)
        if l < len(layer_params) - 1:
            h = _node_update(s[0], s[1], c0, c1, h, root, bias)
        else:
            out = _node_update_out(s[0], s[1], c0, c1, h, root, bias,
                                   lin_out_W, lin_out_b)
    return out


# speedup vs baseline: 3.3400x; 1.6965x over previous
import functools

import jax
import jax.numpy as jnp
from jax import lax
from jax.experimental import pallas as pl
from jax.experimental.pallas import tpu as pltpu
from jax.experimental.pallas import tpu_sc as plsc

_NC = 2
_NS = 16
_NW = _NC * _NS
_CH = 128


def _sc_mesh():
    return plsc.VectorSubcoreMesh(core_axis_name="c", subcore_axis_name="s",
                                  num_cores=_NC, num_subcores=_NS)


_SC_PARAMS = pltpu.CompilerParams(use_tc_tiling_on_sc=False)


def _make_gather(N, E, H):
    epw = E // _NW
    nfull = epw // _CH
    tail = epw - nfull * _CH
    gf = 13 if nfull % 13 == 0 else 1

    @functools.partial(
        pl.kernel,
        mesh=_sc_mesh(),
        out_type=jax.ShapeDtypeStruct((E, H), jnp.float32),
        scratch_types=[
            pltpu.VMEM((epw,), jnp.int32),
            pltpu.VMEM((epw, H), jnp.float32),
            pltpu.SemaphoreType.DMA,
        ],
        compiler_params=_SC_PARAMS,
    )
    def gather_k(h_hbm, src_hbm, out_hbm, idx_v, rows_v, sem):
        wid = lax.axis_index("s") * _NC + lax.axis_index("c")
        base = wid * epw
        pltpu.sync_copy(src_hbm.at[pl.ds(base, epw)], idx_v)

        @pl.loop(0, nfull // gf)
        def _(g):
            descs = []
            for i in range(gf):
                off = (g * gf + i) * _CH
                descs.append(pltpu.async_copy(
                    h_hbm.at[idx_v.at[pl.ds(off, _CH)]],
                    rows_v.at[pl.ds(off, _CH)], sem))
            for d in descs:
                d.wait()

        if tail:
            off = nfull * _CH
            pltpu.async_copy(h_hbm.at[idx_v.at[pl.ds(off, tail)]],
                             rows_v.at[pl.ds(off, tail)], sem).wait()
        pltpu.sync_copy(rows_v, out_hbm.at[pl.ds(base, epw)])

    return gather_k


def _make_scatter(N, E, H):
    epw = E // _NW
    nfull = epw // _CH
    tail = epw - nfull * _CH
    rps = N // _NS

    @functools.partial(
        pl.kernel,
        mesh=_sc_mesh(),
        out_type=jax.ShapeDtypeStruct((_NC, N, H), jnp.float32),
        scratch_types=[
            pltpu.VMEM((epw, H), jnp.float32),
            pltpu.VMEM((nfull, _CH), jnp.int32),
            pltpu.VMEM((tail,), jnp.int32) if tail else None,
            pltpu.VMEM_SHARED((N, H), jnp.float32),
        ],
        compiler_params=_SC_PARAMS,
    )
    def scatter_k(msg_hbm, dst_hbm, out_hbm, vals_v, idx_v, idxt_v, acc_sh):
        cid = lax.axis_index("c")
        sid = lax.axis_index("s")
        wid = sid * _NC + cid
        base = wid * epw

        @pl.loop(0, rps)
        def _(i):
            vals_v[i, :] = jnp.zeros((H,), jnp.float32)

        pltpu.sync_copy(vals_v.at[pl.ds(0, rps)],
                        acc_sh.at[pl.ds(sid * rps, rps)])
        plsc.subcore_barrier()

        pltpu.sync_copy(msg_hbm.at[pl.ds(base, epw)], vals_v)

        @pl.loop(0, nfull)
        def _(j):
            pltpu.sync_copy(dst_hbm.at[pl.ds(base + j * _CH, _CH)],
                            idx_v.at[j])

        if tail:
            pltpu.sync_copy(dst_hbm.at[pl.ds(base + nfull * _CH, tail)],
                            idxt_v)

        @pl.loop(0, nfull)
        def _(j):
            pltpu.sync_copy(vals_v.at[pl.ds(j * _CH, _CH)],
                            acc_sh.at[idx_v.at[j]], add=True)

        if tail:
            pltpu.sync_copy(vals_v.at[pl.ds(nfull * _CH, tail)],
                            acc_sh.at[idxt_v], add=True)
        plsc.subcore_barrier()
        pltpu.sync_copy(acc_sh.at[pl.ds(sid * rps, rps)],
                        out_hbm.at[cid, pl.ds(sid * rps, rps)])

    return scatter_k


def _make_counts(N, E, H):
    epw = E // _NW
    nfull = epw // _CH
    tail = epw - nfull * _CH
    rps = N // _NS

    @functools.partial(
        pl.kernel,
        mesh=_sc_mesh(),
        out_type=jax.ShapeDtypeStruct((_NC, N, H), jnp.float32),
        scratch_types=[
            pltpu.VMEM((max(_CH, rps), H), jnp.float32),
            pltpu.VMEM((nfull, _CH), jnp.int32),
            pltpu.VMEM((tail,), jnp.int32) if tail else None,
            pltpu.VMEM_SHARED((N, H), jnp.float32),
        ],
        compiler_params=_SC_PARAMS,
    )
    def counts_k(dst_hbm, out_hbm, ones_v, idx_v, idxt_v, acc_sh):
        cid = lax.axis_index("c")
        sid = lax.axis_index("s")
        wid = sid * _NC + cid
        base = wid * epw

        @pl.loop(0, max(_CH, rps))
        def _(i):
            ones_v[i, :] = jnp.zeros((H,), jnp.float32)

        pltpu.sync_copy(ones_v.at[pl.ds(0, rps)],
                        acc_sh.at[pl.ds(sid * rps, rps)])
        plsc.subcore_barrier()

        @pl.loop(0, _CH)
        def _(i):
            ones_v[i, :] = jnp.ones((H,), jnp.float32)

        @pl.loop(0, nfull)
        def _(j):
            pltpu.sync_copy(dst_hbm.at[pl.ds(base + j * _CH, _CH)],
                            idx_v.at[j])

        if tail:
            pltpu.sync_copy(dst_hbm.at[pl.ds(base + nfull * _CH, tail)],
                            idxt_v)

        @pl.loop(0, nfull)
        def _(j):
            pltpu.sync_copy(ones_v.at[pl.ds(0, _CH)],
                            acc_sh.at[idx_v.at[j]], add=True)

        if tail:
            pltpu.sync_copy(ones_v.at[pl.ds(0, tail)],
                            acc_sh.at[idxt_v], add=True)
        plsc.subcore_barrier()
        pltpu.sync_copy(acc_sh.at[pl.ds(sid * rps, rps)],
                        out_hbm.at[cid, pl.ds(sid * rps, rps)])

    return counts_k


def _pre_proj(x, W, b, block=2000):
    N, F = x.shape
    H = W.shape[1]

    def body(x_ref, w_ref, b_ref, o_ref):
        o_ref[...] = jnp.maximum(
            jnp.dot(x_ref[...], w_ref[...],
                    preferred_element_type=jnp.float32) + b_ref[...], 0.0)

    return pl.pallas_call(
        body,
        grid=(N // block,),
        in_specs=[
            pl.BlockSpec((block, F), lambda i: (i, 0)),
            pl.BlockSpec((F, H), lambda i: (0, 0)),
            pl.BlockSpec((1, H), lambda i: (0, 0)),
        ],
        out_specs=pl.BlockSpec((block, H), lambda i: (i, 0)),
        out_shape=jax.ShapeDtypeStruct((N, H), jnp.float32),
        compiler_params=pltpu.CompilerParams(
            dimension_semantics=("parallel",)),
    )(x, W, b.reshape(1, H))


def _messages(ea, hs, W1, b1, W2s, B2, block=2000):
    E, FE = ea.shape
    H = hs.shape[1]

    def body(ea_ref, hs_ref, w1_ref, b1_ref, w2_ref, b2_ref, o_ref):
        t = jnp.maximum(
            jnp.dot(ea_ref[...], w1_ref[...],
                    preferred_element_type=jnp.float32) + b1_ref[...], 0.0)
        hs_b = hs_ref[...]
        lane = lax.broadcasted_iota(jnp.int32, (H, H * H), 1)
        sub = lax.broadcasted_iota(jnp.int32, (H, H * H), 0)
        expm = (lane // H == sub).astype(jnp.float32)
        tilem = (lane % H == sub).astype(jnp.float32)
        z = (jnp.dot(t, expm, preferred_element_type=jnp.float32)
             * jnp.dot(hs_b, tilem, preferred_element_type=jnp.float32))
        o_ref[...] = (
            jnp.dot(z, w2_ref[...], preferred_element_type=jnp.float32)
            + jnp.dot(hs_b, b2_ref[...], preferred_element_type=jnp.float32))

    return pl.pallas_call(
        body,
        grid=(E // block,),
        in_specs=[
            pl.BlockSpec((block, FE), lambda i: (i, 0)),
            pl.BlockSpec((block, H), lambda i: (i, 0)),
            pl.BlockSpec((FE, H), lambda i: (0, 0)),
            pl.BlockSpec((1, H), lambda i: (0, 0)),
            pl.BlockSpec((H * H, H), lambda i: (0, 0)),
            pl.BlockSpec((H, H), lambda i: (0, 0)),
        ],
        out_specs=pl.BlockSpec((block, H), lambda i: (i, 0)),
        out_shape=jax.ShapeDtypeStruct((E, H), jnp.float32),
        compiler_params=pltpu.CompilerParams(
            dimension_semantics=("parallel",)),
    )(ea, hs, W1, b1.reshape(1, H), W2s, B2)


def _node_update(s0, s1, c0, c1, h, root, bias):
    N, H = h.shape

    def body(s0_ref, s1_ref, c0_ref, c1_ref, h_ref, r_ref, b_ref, o_ref):
        s = s0_ref[...] + s1_ref[...]
        cnt = jnp.maximum(c0_ref[...] + c1_ref[...], 1.0)
        o_ref[...] = jnp.maximum(
            s / cnt + jnp.dot(h_ref[...], r_ref[...],
                              preferred_element_type=jnp.float32) + b_ref[...],
            0.0)

    return pl.pallas_call(
        body,
        out_shape=jax.ShapeDtypeStruct((N, H), jnp.float32),
    )(s0, s1, c0, c1, h, root, bias.reshape(1, H))


def _node_update_out(s0, s1, c0, c1, h, root, bias, Wout, bout):
    N, H = h.shape
    OUT = Wout.shape[1]

    def body(s0_ref, s1_ref, c0_ref, c1_ref, h_ref, r_ref, b_ref,
             wo_ref, bo_ref, o_ref):
        s = s0_ref[...] + s1_ref[...]
        cnt = jnp.maximum(c0_ref[...] + c1_ref[...], 1.0)
        hn = jnp.maximum(
            s / cnt + jnp.dot(h_ref[...], r_ref[...],
                              preferred_element_type=jnp.float32) + b_ref[...],
            0.0)
        o_ref[...] = jnp.dot(hn, wo_ref[...],
                             preferred_element_type=jnp.float32) + bo_ref[...]

    return pl.pallas_call(
        body,
        out_shape=jax.ShapeDtypeStruct((N, OUT), jnp.float32),
    )(s0, s1, c0, c1, h, root, bias.reshape(1, H), Wout, bout.reshape(1, OUT))


def kernel(x, edge_index, edge_attr, lin_in_W, lin_in_b,
           mlp0_W1, mlp0_b1, mlp0_W2, mlp0_b2, root0, bias0,
           mlp1_W1, mlp1_b1, mlp1_W2, mlp1_b2, root1, bias1,
           mlp2_W1, mlp2_b1, mlp2_W2, mlp2_b2, root2, bias2,
           lin_out_W, lin_out_b):
    N, _ = x.shape
    E = edge_index.shape[1]
    H = lin_in_W.shape[1]

    src = edge_index[0]
    dst = edge_index[1]

    gather_k = _make_gather(N, E, H)
    scatter_k = _make_scatter(N, E, H)
    counts_k = _make_counts(N, E, H)

    layer_params = [(mlp0_W1, mlp0_b1, mlp0_W2, mlp0_b2, root0, bias0),
                    (mlp1_W1, mlp1_b1, mlp1_W2, mlp1_b2, root1, bias1),
                    (mlp2_W1, mlp2_b1, mlp2_W2, mlp2_b2, root2, bias2)]

    h = _pre_proj(x, lin_in_W, lin_in_b)
    cnt = counts_k(dst)
    c0, c1 = cnt[0], cnt[1]

    out = None
    for l, (W1, b1, W2, b2, root, bias) in enumerate(layer_params):
        hs = gather_k(h, src)
        msg = _messages(edge_attr, hs, W1, b1, W2.reshape(H * H, H),
                        b2.reshape(H, H))
        s = scatter_k(msg, dst)
        if l < len(layer_params) - 1:
            h = _node_update(s[0], s[1], c0, c1, h, root, bias)
        else:
            out = _node_update_out(s[0], s[1], c0, c1, h, root, bias,
                                   lin_out_W, lin_out_b)
    return out


# speedup vs baseline: 5.1447x; 1.5403x over previous
import functools

import jax
import jax.numpy as jnp
from jax import lax
from jax.experimental import pallas as pl
from jax.experimental.pallas import tpu as pltpu
from jax.experimental.pallas import tpu_sc as plsc

_NC = 2
_NS = 16
_NW = _NC * _NS
_CH = 128
_P = 8


def _sc_mesh():
    return plsc.VectorSubcoreMesh(core_axis_name="c", subcore_axis_name="s",
                                  num_cores=_NC, num_subcores=_NS)


_SC_PARAMS = pltpu.CompilerParams(use_tc_tiling_on_sc=False)


def _make_gather(N, E, H):
    epw = E // _NW
    nfull = epw // _CH
    tail = epw - nfull * _CH
    gf = 13 if nfull % 13 == 0 else 1

    @functools.partial(
        pl.kernel,
        mesh=_sc_mesh(),
        out_type=jax.ShapeDtypeStruct((E, H), jnp.float32),
        scratch_types=[
            pltpu.VMEM((epw,), jnp.int32),
            pltpu.VMEM((epw, H), jnp.float32),
            pltpu.SemaphoreType.DMA,
        ],
        compiler_params=_SC_PARAMS,
    )
    def gather_k(h_hbm, src_hbm, out_hbm, idx_v, rows_v, sem):
        wid = lax.axis_index("s") * _NC + lax.axis_index("c")
        base = wid * epw
        pltpu.sync_copy(src_hbm.at[pl.ds(base, epw)], idx_v)

        @pl.loop(0, nfull // gf)
        def _(g):
            descs = []
            for i in range(gf):
                off = (g * gf + i) * _CH
                descs.append(pltpu.async_copy(
                    h_hbm.at[idx_v.at[pl.ds(off, _CH)]],
                    rows_v.at[pl.ds(off, _CH)], sem))
            for d in descs:
                d.wait()

        if tail:
            off = nfull * _CH
            pltpu.async_copy(h_hbm.at[idx_v.at[pl.ds(off, tail)]],
                             rows_v.at[pl.ds(off, tail)], sem).wait()
        pltpu.sync_copy(rows_v, out_hbm.at[pl.ds(base, epw)])

    return gather_k


def _make_scatter(N, E, H):
    epw = E // _NW
    nfull = epw // _CH
    tail = epw - nfull * _CH
    rps = N // _NS

    @functools.partial(
        pl.kernel,
        mesh=_sc_mesh(),
        out_type=jax.ShapeDtypeStruct((_NC, N, H), jnp.float32),
        scratch_types=[
            pltpu.VMEM((epw, H), jnp.float32),
            pltpu.VMEM((nfull, _CH), jnp.int32),
            pltpu.VMEM((tail,), jnp.int32),
            pltpu.VMEM_SHARED((N, H), jnp.float32),
        ],
        compiler_params=_SC_PARAMS,
    )
    def scatter_k(msg_hbm, dst_hbm, out_hbm, vals_v, idx_v, idxt_v, acc_sh):
        cid = lax.axis_index("c")
        sid = lax.axis_index("s")
        wid = sid * _NC + cid
        base = wid * epw

        @pl.loop(0, rps)
        def _(i):
            vals_v[i, :] = jnp.zeros((H,), jnp.float32)

        pltpu.sync_copy(vals_v.at[pl.ds(0, rps)],
                        acc_sh.at[pl.ds(sid * rps, rps)])
        plsc.subcore_barrier()

        pltpu.sync_copy(msg_hbm.at[pl.ds(base, epw)], vals_v)

        @pl.loop(0, nfull)
        def _(j):
            pltpu.sync_copy(dst_hbm.at[pl.ds(base + j * _CH, _CH)],
                            idx_v.at[j])

        pltpu.sync_copy(dst_hbm.at[pl.ds(base + nfull * _CH, tail)], idxt_v)

        @pl.loop(0, nfull)
        def _(j):
            pltpu.sync_copy(vals_v.at[pl.ds(j * _CH, _CH)],
                            acc_sh.at[idx_v.at[j]], add=True)

        pltpu.sync_copy(vals_v.at[pl.ds(nfull * _CH, tail)],
                        acc_sh.at[idxt_v], add=True)
        plsc.subcore_barrier()
        pltpu.sync_copy(acc_sh.at[pl.ds(sid * rps, rps)],
                        out_hbm.at[cid, pl.ds(sid * rps, rps)])

    return scatter_k


def _make_counts(N, E, H):
    epw = E // _NW
    nfull = epw // _CH
    tail = epw - nfull * _CH
    rps = N // _NS

    @functools.partial(
        pl.kernel,
        mesh=_sc_mesh(),
        out_type=jax.ShapeDtypeStruct((_NC, N, H), jnp.float32),
        scratch_types=[
            pltpu.VMEM((max(_CH, rps), H), jnp.float32),
            pltpu.VMEM((nfull, _CH), jnp.int32),
            pltpu.VMEM((tail,), jnp.int32),
            pltpu.VMEM_SHARED((N, H), jnp.float32),
        ],
        compiler_params=_SC_PARAMS,
    )
    def counts_k(dst_hbm, out_hbm, ones_v, idx_v, idxt_v, acc_sh):
        cid = lax.axis_index("c")
        sid = lax.axis_index("s")
        wid = sid * _NC + cid
        base = wid * epw

        @pl.loop(0, max(_CH, rps))
        def _(i):
            ones_v[i, :] = jnp.zeros((H,), jnp.float32)

        pltpu.sync_copy(ones_v.at[pl.ds(0, rps)],
                        acc_sh.at[pl.ds(sid * rps, rps)])
        plsc.subcore_barrier()

        @pl.loop(0, _CH)
        def _(i):
            ones_v[i, :] = jnp.ones((H,), jnp.float32)

        @pl.loop(0, nfull)
        def _(j):
            pltpu.sync_copy(dst_hbm.at[pl.ds(base + j * _CH, _CH)],
                            idx_v.at[j])

        pltpu.sync_copy(dst_hbm.at[pl.ds(base + nfull * _CH, tail)], idxt_v)

        @pl.loop(0, nfull)
        def _(j):
            pltpu.sync_copy(ones_v.at[pl.ds(0, _CH)],
                            acc_sh.at[idx_v.at[j]], add=True)

        pltpu.sync_copy(ones_v.at[pl.ds(0, tail)],
                        acc_sh.at[idxt_v], add=True)
        plsc.subcore_barrier()
        pltpu.sync_copy(acc_sh.at[pl.ds(sid * rps, rps)],
                        out_hbm.at[cid, pl.ds(sid * rps, rps)])

    return counts_k


def _pre_proj(xw, Wbd, bt):
    Nb, F8 = xw.shape
    HP = Wbd.shape[1]

    def body(x_ref, w_ref, b_ref, o_ref):
        o_ref[...] = jnp.maximum(
            jnp.dot(x_ref[...], w_ref[...],
                    preferred_element_type=jnp.float32) + b_ref[...], 0.0)

    return pl.pallas_call(
        body,
        out_shape=jax.ShapeDtypeStruct((Nb, HP), jnp.float32),
    )(xw, Wbd, bt)


def _messages(ea_p, hs_p, W1, b1, W2p, B2, H, block=1000):
    Eb = ea_p.shape[0]

    def body(ea_ref, hs_ref, w1_ref, b1_ref, w2_ref, b2_ref, o_ref):
        lane = lax.broadcasted_iota(jnp.int32, (H, H * H), 1)
        sub = lax.broadcasted_iota(jnp.int32, (H, H * H), 0)
        expm = (lane // H == sub).astype(jnp.float32)
        tilem = (lane % H == sub).astype(jnp.float32)
        for j in range(_P):
            ea_j = ea_ref[:, j * H:(j + 1) * H]
            hs_j = hs_ref[:, j * H:(j + 1) * H]
            t = jnp.maximum(
                jnp.dot(ea_j, w1_ref[...],
                        preferred_element_type=jnp.float32) + b1_ref[...], 0.0)
            z = (jnp.dot(t, expm, preferred_element_type=jnp.float32)
                 * jnp.dot(hs_j, tilem, preferred_element_type=jnp.float32))
            o_ref[:, j * H:(j + 1) * H] = (
                jnp.dot(z, w2_ref[...], preferred_element_type=jnp.float32)
                + jnp.dot(hs_j, b2_ref[...],
                          preferred_element_type=jnp.float32))

    return pl.pallas_call(
        body,
        grid=(Eb // block,),
        in_specs=[
            pl.BlockSpec((block, _P * H), lambda i: (i, 0)),
            pl.BlockSpec((block, _P * H), lambda i: (i, 0)),
            pl.BlockSpec((H, H), lambda i: (0, 0)),
            pl.BlockSpec((1, H), lambda i: (0, 0)),
            pl.BlockSpec((H * H, H), lambda i: (0, 0)),
            pl.BlockSpec((H, H), lambda i: (0, 0)),
        ],
        out_specs=pl.BlockSpec((block, _P * H), lambda i: (i, 0)),
        out_shape=jax.ShapeDtypeStruct((Eb, _P * H), jnp.float32),
        compiler_params=pltpu.CompilerParams(
            dimension_semantics=("parallel",)),
    )(ea_p, hs_p, W1, b1.reshape(1, H), W2p, B2)


def _node_update(s0, s1, c0, c1, h_p, root_bd, bias_t):
    Nb, HP = h_p.shape

    def body(s0_ref, s1_ref, c0_ref, c1_ref, h_ref, r_ref, b_ref, o_ref):
        s = s0_ref[...] + s1_ref[...]
        cnt = jnp.maximum(c0_ref[...] + c1_ref[...], 1.0)
        o_ref[...] = jnp.maximum(
            s / cnt + jnp.dot(h_ref[...], r_ref[...],
                              preferred_element_type=jnp.float32) + b_ref[...],
            0.0)

    return pl.pallas_call(
        body,
        out_shape=jax.ShapeDtypeStruct((Nb, HP), jnp.float32),
    )(s0, s1, c0, c1, h_p, root_bd, bias_t)


def _node_update_out(s0, s1, c0, c1, h_p, root_bd, bias_t, Wout_bd, bout_t):
    Nb, HP = h_p.shape
    OP = Wout_bd.shape[1]

    def body(s0_ref, s1_ref, c0_ref, c1_ref, h_ref, r_ref, b_ref,
             wo_ref, bo_ref, o_ref):
        s = s0_ref[...] + s1_ref[...]
        cnt = jnp.maximum(c0_ref[...] + c1_ref[...], 1.0)
        hn = jnp.maximum(
            s / cnt + jnp.dot(h_ref[...], r_ref[...],
                              preferred_element_type=jnp.float32) + b_ref[...],
            0.0)
        o_ref[...] = jnp.dot(hn, wo_ref[...],
                             preferred_element_type=jnp.float32) + bo_ref[...]

    return pl.pallas_call(
        body,
        out_shape=jax.ShapeDtypeStruct((Nb, OP), jnp.float32),
    )(s0, s1, c0, c1, h_p, root_bd, bias_t, Wout_bd, bout_t)


def kernel(x, edge_index, edge_attr, lin_in_W, lin_in_b,
           mlp0_W1, mlp0_b1, mlp0_W2, mlp0_b2, root0, bias0,
           mlp1_W1, mlp1_b1, mlp1_W2, mlp1_b2, root1, bias1,
           mlp2_W1, mlp2_b1, mlp2_W2, mlp2_b2, root2, bias2,
           lin_out_W, lin_out_b):
    N, F_IN = x.shape
    E = edge_index.shape[1]
    H = lin_in_W.shape[1]
    OUT = lin_out_W.shape[1]
    Nb, Eb = N // _P, E // _P

    src = edge_index[0]
    dst = edge_index[1]
    ea_p = edge_attr.reshape(Eb, _P * H)
    eye8 = jnp.eye(_P, dtype=jnp.float32)

    gather_k = _make_gather(N, E, H)
    scatter_k = _make_scatter(N, E, H)
    counts_k = _make_counts(N, E, H)

    layer_params = [(mlp0_W1, mlp0_b1, mlp0_W2, mlp0_b2, root0, bias0),
                    (mlp1_W1, mlp1_b1, mlp1_W2, mlp1_b2, root1, bias1),
                    (mlp2_W1, mlp2_b1, mlp2_W2, mlp2_b2, root2, bias2)]

    h_p = _pre_proj(x.reshape(Nb, _P * F_IN), jnp.kron(eye8, lin_in_W),
                    jnp.tile(lin_in_b, _P).reshape(1, _P * H))
    cnt = counts_k(dst)
    c0 = cnt[0].reshape(Nb, _P * H)
    c1 = cnt[1].reshape(Nb, _P * H)

    out = None
    for l, (W1, b1, W2, b2, root, bias) in enumerate(layer_params):
        hs = gather_k(h_p.reshape(N, H), src)
        msg_p = _messages(ea_p, hs.reshape(Eb, _P * H), W1, b1,
                          W2.reshape(H * H, H), b2.reshape(H, H), H)
        s = scatter_k(msg_p.reshape(E, H), dst)
        s0 = s[0].reshape(Nb, _P * H)
        s1 = s[1].reshape(Nb, _P * H)
        root_bd = jnp.kron(eye8, root)
        bias_t = jnp.tile(bias, _P).reshape(1, _P * H)
        if l < len(layer_params) - 1:
            h_p = _node_update(s0, s1, c0, c1, h_p, root_bd, bias_t)
        else:
            out = _node_update_out(
                s0, s1, c0, c1, h_p, root_bd, bias_t,
                jnp.kron(eye8, lin_out_W),
                jnp.tile(lin_out_b, _P).reshape(1, _P * OUT))
    return out.reshape(N, OUT)
